# baseline (speedup 1.0000x reference)
import functools

import jax
import jax.numpy as jnp
from jax import lax
from jax.experimental import pallas as pl
from jax.experimental.pallas import tpu as pltpu
from jax.experimental.pallas import tpu_sc as plsc

BATCH = 16384
HIST = 20
DIM = 32
TOTAL = BATCH * HIST

NC = 2
NS = 16
NW = NC * NS
BPW = TOTAL // NW

CHUNK = 128
NBUF = 4
NCH = BPW // CHUNK
CPH = BATCH // CHUNK
NPIECE = 10
PIECE = BPW // NPIECE

_mesh = plsc.VectorSubcoreMesh(core_axis_name="c", subcore_axis_name="s")


@functools.partial(
    pl.kernel,
    mesh=_mesh,
    out_type=jax.ShapeDtypeStruct((HIST, DIM // 8, CPH, 8, CHUNK), jnp.float32),
    scratch_types=(
        [
            pltpu.VMEM((BPW,), jnp.int32),
            pltpu.VMEM((NBUF, CHUNK, DIM), jnp.float32),
            pltpu.VMEM((NBUF, DIM, CHUNK), jnp.float32),
        ]
        + [pltpu.SemaphoreType.DMA] * (1 + 2 * NBUF)
    ),
    compiler_params=pltpu.CompilerParams(
        use_tc_tiling_on_sc=False, needs_layout_passes=False
    ),
)
def _gather(xs_hbm, table_hbm, out_hbm, idx_v, rows_v, tbuf_v, *sems):
    wid = lax.axis_index("s") * NC + lax.axis_index("c")
    isem = sems[0]
    gsem = sems[1 : 1 + NBUF]
    wsem = sems[1 + NBUF :]
    iota16 = lax.iota(jnp.int32, 16)

    for p in range(NPIECE):
        f = wid * BPW + p * PIECE
        pltpu.async_copy(
            xs_hbm.at[f // BATCH, pl.ds(f % BATCH, PIECE)],
            idx_v.at[pl.ds(p * PIECE, PIECE)],
            isem,
        )
    for p in range(NPIECE):
        pltpu.make_async_copy(
            xs_hbm.at[0, pl.ds(0, PIECE)],
            idx_v.at[pl.ds(p * PIECE, PIECE)],
            isem,
        ).wait()

    def chunk_pos(j):
        c = wid * NCH + j
        return c // CPH, c % CPH

    def fire_gather(b, j):
        pltpu.async_copy(
            table_hbm.at[idx_v.at[pl.ds(j * CHUNK, CHUNK)]],
            rows_v.at[b],
            gsem[b],
        )

    def drain_gather(b):
        pltpu.make_async_copy(
            table_hbm.at[pl.ds(0, CHUNK)], rows_v.at[b], gsem[b]
        ).wait()

    def transpose(b):
        rv = rows_v.at[b]
        tb = tbuf_v.at[b]
        for d in range(DIM):
            for jj in range(8):
                v = plsc.load_gather(
                    rv, [iota16 + jj * 16, jnp.full((16,), d, jnp.int32)]
                )
                tb[d, pl.ds(jj * 16, 16)] = v

    def fire_wb(b, j):
        h, tbc = chunk_pos(j)
        for td in range(DIM // 8):
            pltpu.async_copy(
                tbuf_v.at[b, pl.ds(td * 8, 8)], out_hbm.at[h, td, tbc], wsem[b]
            )

    def drain_wb(b):
        for td in range(DIM // 8):
            pltpu.make_async_copy(
                tbuf_v.at[b, pl.ds(td * 8, 8)], out_hbm.at[0, td, 0], wsem[b]
            ).wait()

    for b in range(NBUF):
        fire_gather(b, b)
    for b in range(NBUF):
        drain_gather(b)
        transpose(b)
        fire_wb(b, b)
        fire_gather(b, b + NBUF)

    def body(j0, carry):
        for b in range(NBUF):
            j = j0 * NBUF + b
            drain_gather(b)
            drain_wb(b)
            transpose(b)
            fire_wb(b, j)
            fire_gather(b, j + NBUF)
        return carry

    lax.fori_loop(1, NCH // NBUF - 1, body, 0)

    for b in range(NBUF):
        j = NCH - NBUF + b
        drain_gather(b)
        drain_wb(b)
        transpose(b)
        fire_wb(b, j)
    for b in range(NBUF):
        drain_wb(b)


def kernel(xs, table):
    out_t = _gather(xs.T.astype(jnp.int32), table)
    out = jnp.transpose(out_t, (2, 4, 0, 1, 3))
    return out.reshape(BATCH, HIST, DIM)


# speedup vs baseline: 1.4356x; 1.4356x over previous
import functools

import jax
import jax.numpy as jnp
from jax import lax
from jax.experimental import pallas as pl
from jax.experimental.pallas import tpu as pltpu
from jax.experimental.pallas import tpu_sc as plsc

BATCH = 16384
HIST = 20
DIM = 32
TOTAL = BATCH * HIST

NC = 2
NS = 16
NW = NC * NS
BPW = TOTAL // NW

CHUNK = 128
NBUF = 4
NCH = BPW // CHUNK
CPH = BATCH // CHUNK
NPIECE = 10
PIECE = BPW // NPIECE

_mesh = plsc.VectorSubcoreMesh(core_axis_name="c", subcore_axis_name="s")


@functools.partial(
    pl.kernel,
    mesh=_mesh,
    out_type=jax.ShapeDtypeStruct((HIST, DIM // 8, CPH, 8, CHUNK), jnp.float32),
    scratch_types=(
        [
            pltpu.VMEM((BPW,), jnp.int32),
            pltpu.VMEM((NBUF, CHUNK, DIM), jnp.float32),
            pltpu.VMEM((NBUF, DIM, CHUNK), jnp.float32),
        ]
        + [pltpu.SemaphoreType.DMA] * (1 + 2 * NBUF)
    ),
    compiler_params=pltpu.CompilerParams(
        use_tc_tiling_on_sc=False, needs_layout_passes=False
    ),
)
def _gather(xs_hbm, table_hbm, out_hbm, idx_v, rows_v, tbuf_v, *sems):
    wid = lax.axis_index("s") * NC + lax.axis_index("c")
    isem = sems[0]
    gsem = sems[1 : 1 + NBUF]
    wsem = sems[1 + NBUF :]
    iota16 = lax.iota(jnp.int32, 16)

    for p in range(NPIECE):
        f = wid * BPW + p * PIECE
        pltpu.async_copy(
            xs_hbm.at[f // BATCH, pl.ds(f % BATCH, PIECE)],
            idx_v.at[pl.ds(p * PIECE, PIECE)],
            isem,
        )
    for p in range(NPIECE):
        pltpu.make_async_copy(
            xs_hbm.at[0, pl.ds(0, PIECE)],
            idx_v.at[pl.ds(p * PIECE, PIECE)],
            isem,
        ).wait()

    def chunk_pos(j):
        c = wid * NCH + j
        return c // CPH, c % CPH

    def fire_gather(b, j):
        pltpu.async_copy(
            table_hbm.at[idx_v.at[pl.ds(j * CHUNK, CHUNK)]],
            rows_v.at[b],
            gsem[b],
        )

    def drain_gather(b):
        pltpu.make_async_copy(
            table_hbm.at[pl.ds(0, CHUNK)], rows_v.at[b], gsem[b]
        ).wait()

    def transpose(b):
        rv = rows_v.at[b]
        tb = tbuf_v.at[b]

        @plsc.parallel_loop(0, DIM, step=1, unroll=4)
        def _(d):
            for jj in range(8):
                v = plsc.load_gather(
                    rv, [iota16 + jj * 16, jnp.full((16,), d, jnp.int32)]
                )
                tb[d, pl.ds(jj * 16, 16)] = v

    def fire_wb(b, j):
        h, tbc = chunk_pos(j)
        for td in range(DIM // 8):
            pltpu.async_copy(
                tbuf_v.at[b, pl.ds(td * 8, 8)], out_hbm.at[h, td, tbc], wsem[b]
            )

    def drain_wb(b):
        for td in range(DIM // 8):
            pltpu.make_async_copy(
                tbuf_v.at[b, pl.ds(td * 8, 8)], out_hbm.at[0, td, 0], wsem[b]
            ).wait()

    for b in range(NBUF):
        fire_gather(b, b)
    for b in range(NBUF):
        drain_gather(b)
        transpose(b)
        fire_wb(b, b)
        fire_gather(b, b + NBUF)

    def body(j0, carry):
        for b in range(NBUF):
            j = j0 * NBUF + b
            drain_gather(b)
            drain_wb(b)
            transpose(b)
            fire_wb(b, j)
            fire_gather(b, j + NBUF)
        return carry

    lax.fori_loop(1, NCH // NBUF - 1, body, 0)

    for b in range(NBUF):
        j = NCH - NBUF + b
        drain_gather(b)
        drain_wb(b)
        transpose(b)
        fire_wb(b, j)
    for b in range(NBUF):
        drain_wb(b)


NVOC = 1000000
SLAB = 128
NSLAB = NVOC // SLAB
MAIN = NSLAB // NW
FNBUF = 4


@functools.partial(
    pl.kernel,
    mesh=_mesh,
    out_type=jax.ShapeDtypeStruct((NVOC * DIM,), jnp.float32),
    scratch_types=(
        [
            pltpu.VMEM((FNBUF, DIM // 8, 8, SLAB), jnp.float32),
            pltpu.VMEM((FNBUF * SLAB * DIM,), jnp.float32),
            pltpu.VMEM((64 * DIM,), jnp.float32),
        ]
        + [pltpu.SemaphoreType.DMA] * (2 * FNBUF)
    ),
    compiler_params=pltpu.CompilerParams(
        use_tc_tiling_on_sc=True, needs_layout_passes=False
    ),
)
def _format(tab_hbm, tail_hbm, out_hbm, buf_v, tbuf_v, tail_v, *sems):
    wid = lax.axis_index("s") * NC + lax.axis_index("c")
    rsem = sems[:FNBUF]
    wsem = sems[FNBUF:]
    iota16 = lax.iota(jnp.int32, 16)
    th_lo = iota16 // 8
    hi_v = iota16 % 8

    def slab_v0(m):
        return (m * NW + wid) * SLAB

    def fire_read(b, m):
        v0 = slab_v0(m)
        for td in range(DIM // 8):
            pltpu.async_copy(
                tab_hbm.at[pl.ds(td * 8, 8), pl.ds(v0, SLAB)],
                buf_v.at[b, td],
                rsem[b],
            )

    def drain_read(b):
        for td in range(DIM // 8):
            pltpu.make_async_copy(
                tab_hbm.at[pl.ds(0, 8), pl.ds(0, SLAB)], buf_v.at[b, td], rsem[b]
            ).wait()

    def transpose_slab(b):
        bv = buf_v.at[b]
        base = b * SLAB * DIM

        @plsc.parallel_loop(0, SLAB, step=1, unroll=4)
        def _(vi):
            for half in range(2):
                v = plsc.load_gather(
                    bv, [th_lo + 2 * half, hi_v, jnp.full((16,), vi, jnp.int32)]
                )
                tbuf_v[pl.ds(base + vi * DIM + half * 16, 16)] = v

    def fire_write(b, m):
        v0 = slab_v0(m)
        pltpu.async_copy(
            tbuf_v.at[pl.ds(b * SLAB * DIM, SLAB * DIM)],
            out_hbm.at[pl.ds(v0 * DIM, SLAB * DIM)],
            wsem[b],
        )

    def drain_write(b):
        pltpu.make_async_copy(
            tbuf_v.at[pl.ds(b * SLAB * DIM, SLAB * DIM)],
            out_hbm.at[pl.ds(0, SLAB * DIM)],
            wsem[b],
        ).wait()

    for b in range(FNBUF):
        fire_read(b, b)
    for b in range(FNBUF):
        drain_read(b)
        transpose_slab(b)
        fire_write(b, b)
        fire_read(b, b + FNBUF)

    def body(p, carry):
        for b in range(FNBUF):
            m = p * FNBUF + b
            drain_read(b)
            drain_write(b)
            transpose_slab(b)
            fire_write(b, m)
            fire_read(b, m + FNBUF)
        return carry

    lax.fori_loop(1, MAIN // FNBUF - 1, body, 0)

    for b in range(FNBUF):
        m = MAIN - FNBUF + b
        drain_read(b)
        drain_write(b)
        transpose_slab(b)
        fire_write(b, m)
    for b in range(FNBUF):
        drain_write(b)

    @pl.when(wid < NSLAB - MAIN * NW)
    def _():
        v0 = (MAIN * NW + wid) * SLAB
        for td in range(DIM // 8):
            pltpu.async_copy(
                tab_hbm.at[pl.ds(td * 8, 8), pl.ds(v0, SLAB)],
                buf_v.at[0, td],
                rsem[0],
            )
        drain_read(0)
        transpose_slab(0)
        pltpu.async_copy(
            tbuf_v.at[pl.ds(0, SLAB * DIM)],
            out_hbm.at[pl.ds(v0 * DIM, SLAB * DIM)],
            wsem[0],
        )
        drain_write(0)

    @pl.when(wid == 4)
    def _():
        v0 = NSLAB * SLAB
        pltpu.async_copy(tail_hbm, tail_v, rsem[1])
        pltpu.make_async_copy(tail_hbm, tail_v, rsem[1]).wait()
        pltpu.async_copy(
            tail_v, out_hbm.at[pl.ds(v0 * DIM, 64 * DIM)], wsem[1]
        )
        pltpu.make_async_copy(
            tail_v, out_hbm.at[pl.ds(0, 64 * DIM)], wsem[1]
        ).wait()


def kernel(xs, table):
    tail_lin = table[NSLAB * SLAB :].reshape(64 * DIM)
    tab_lin = _format(table.T, tail_lin)
    out_t = _gather(xs.T.astype(jnp.int32), tab_lin.reshape(NVOC, DIM))
    out = jnp.transpose(out_t, (2, 4, 0, 1, 3))
    return out.reshape(BATCH, HIST, DIM)


# speedup vs baseline: 1.4461x; 1.0074x over previous
import functools

import jax
import jax.numpy as jnp
from jax import lax
from jax.experimental import pallas as pl
from jax.experimental.pallas import tpu as pltpu
from jax.experimental.pallas import tpu_sc as plsc

BATCH = 16384
HIST = 20
DIM = 32
TOTAL = BATCH * HIST

NC = 2
NS = 16
NW = NC * NS
BPW = TOTAL // NW

CHUNK = 128
NBUF = 4
NCH = BPW // CHUNK
CPH = BATCH // CHUNK
NPIECE = 10
PIECE = BPW // NPIECE

_mesh = plsc.VectorSubcoreMesh(core_axis_name="c", subcore_axis_name="s")


@functools.partial(
    pl.kernel,
    mesh=_mesh,
    out_type=jax.ShapeDtypeStruct((HIST, DIM // 8, CPH, 8, CHUNK), jnp.float32),
    scratch_types=(
        [
            pltpu.VMEM((BPW,), jnp.int32),
            pltpu.VMEM((NBUF, CHUNK, DIM), jnp.float32),
            pltpu.VMEM((NBUF, DIM // 8, 8, CHUNK), jnp.float32),
        ]
        + [pltpu.SemaphoreType.DMA] * (1 + 2 * NBUF)
    ),
    compiler_params=pltpu.CompilerParams(
        use_tc_tiling_on_sc=False, needs_layout_passes=False
    ),
)
def _gather(xs_hbm, table_hbm, out_hbm, idx_v, rows_v, tbuf_v, *sems):
    wid = lax.axis_index("s") * NC + lax.axis_index("c")
    isem = sems[0]
    gsem = sems[1 : 1 + NBUF]
    wsem = sems[1 + NBUF :]
    iota16 = lax.iota(jnp.int32, 16)

    for p in range(NPIECE):
        f = wid * BPW + p * PIECE
        pltpu.async_copy(
            xs_hbm.at[f // BATCH, pl.ds(f % BATCH, PIECE)],
            idx_v.at[pl.ds(p * PIECE, PIECE)],
            isem,
        )
    for p in range(NPIECE):
        pltpu.make_async_copy(
            xs_hbm.at[0, pl.ds(0, PIECE)],
            idx_v.at[pl.ds(p * PIECE, PIECE)],
            isem,
        ).wait()

    def chunk_pos(j):
        c = wid * NCH + j
        return c // CPH, c % CPH

    def fire_gather(b, j):
        pltpu.async_copy(
            table_hbm.at[idx_v.at[pl.ds(j * CHUNK, CHUNK)]],
            rows_v.at[b],
            gsem[b],
        )

    def drain_gather(b):
        pltpu.make_async_copy(
            table_hbm.at[pl.ds(0, CHUNK)], rows_v.at[b], gsem[b]
        ).wait()

    def transpose(b):
        rv = rows_v.at[b]
        tb = tbuf_v.at[b]

        @plsc.parallel_loop(0, DIM, step=1, unroll=4)
        def _(d):
            for jj in range(8):
                v = plsc.load_gather(
                    rv, [iota16 + jj * 16, jnp.full((16,), d, jnp.int32)]
                )
                tb[d // 8, d % 8, pl.ds(jj * 16, 16)] = v

    def fire_wb(b, j):
        h, tbc = chunk_pos(j)
        pltpu.async_copy(
            tbuf_v.at[b], out_hbm.at[h].at[:, tbc], wsem[b]
        )

    def drain_wb(b):
        pltpu.make_async_copy(
            tbuf_v.at[b], out_hbm.at[0].at[:, 0], wsem[b]
        ).wait()

    for b in range(NBUF):
        fire_gather(b, b)
    for b in range(NBUF):
        drain_gather(b)
        transpose(b)
        fire_wb(b, b)
        fire_gather(b, b + NBUF)

    def body(j0, carry):
        for b in range(NBUF):
            j = j0 * NBUF + b
            drain_gather(b)
            drain_wb(b)
            transpose(b)
            fire_wb(b, j)
            fire_gather(b, j + NBUF)
        return carry

    lax.fori_loop(1, NCH // NBUF - 1, body, 0)

    for b in range(NBUF):
        j = NCH - NBUF + b
        drain_gather(b)
        drain_wb(b)
        transpose(b)
        fire_wb(b, j)
    for b in range(NBUF):
        drain_wb(b)


NVOC = 1000000
SLAB = 128
NSLAB = NVOC // SLAB
MAIN = NSLAB // NW
FNBUF = 4


@functools.partial(
    pl.kernel,
    mesh=_mesh,
    out_type=jax.ShapeDtypeStruct((NVOC * DIM,), jnp.float32),
    scratch_types=(
        [
            pltpu.VMEM((FNBUF, DIM, SLAB), jnp.float32),
            pltpu.VMEM((FNBUF * SLAB * DIM,), jnp.float32),
            pltpu.VMEM((64 * DIM,), jnp.float32),
        ]
        + [pltpu.SemaphoreType.DMA] * (2 * FNBUF)
    ),
    compiler_params=pltpu.CompilerParams(
        use_tc_tiling_on_sc=True, needs_layout_passes=False
    ),
)
def _format(tab_hbm, tail_hbm, out_hbm, buf_v, tbuf_v, tail_v, *sems):
    wid = lax.axis_index("s") * NC + lax.axis_index("c")
    rsem = sems[:FNBUF]
    wsem = sems[FNBUF:]
    iota16 = lax.iota(jnp.int32, 16)
    th_lo = iota16 // 8
    hi_v = iota16 % 8

    def slab_v0(m):
        return (m * NW + wid) * SLAB

    def fire_read(b, m):
        v0 = slab_v0(m)
        pltpu.async_copy(
            tab_hbm.at[:, pl.ds(v0, SLAB)], buf_v.at[b], rsem[b]
        )

    def drain_read(b):
        pltpu.make_async_copy(
            tab_hbm.at[:, pl.ds(0, SLAB)], buf_v.at[b], rsem[b]
        ).wait()

    def transpose_slab(b):
        bv = buf_v.at[b]
        base = b * SLAB * DIM

        @plsc.parallel_loop(0, SLAB, step=1, unroll=4)
        def _(vi):
            for half in range(2):
                v = plsc.load_gather(
                    bv, [iota16 + half * 16, jnp.full((16,), vi, jnp.int32)]
                )
                tbuf_v[pl.ds(base + vi * DIM + half * 16, 16)] = v

    def fire_write(b, m):
        v0 = slab_v0(m)
        pltpu.async_copy(
            tbuf_v.at[pl.ds(b * SLAB * DIM, SLAB * DIM)],
            out_hbm.at[pl.ds(v0 * DIM, SLAB * DIM)],
            wsem[b],
        )

    def drain_write(b):
        pltpu.make_async_copy(
            tbuf_v.at[pl.ds(b * SLAB * DIM, SLAB * DIM)],
            out_hbm.at[pl.ds(0, SLAB * DIM)],
            wsem[b],
        ).wait()

    for b in range(FNBUF):
        fire_read(b, b)
    for b in range(FNBUF):
        drain_read(b)
        transpose_slab(b)
        fire_write(b, b)
        fire_read(b, b + FNBUF)

    def body(p, carry):
        for b in range(FNBUF):
            m = p * FNBUF + b
            drain_read(b)
            drain_write(b)
            transpose_slab(b)
            fire_write(b, m)
            fire_read(b, m + FNBUF)
        return carry

    lax.fori_loop(1, MAIN // FNBUF - 1, body, 0)

    for b in range(FNBUF):
        m = MAIN - FNBUF + b
        drain_read(b)
        drain_write(b)
        transpose_slab(b)
        fire_write(b, m)
    for b in range(FNBUF):
        drain_write(b)

    @pl.when(wid < NSLAB - MAIN * NW)
    def _():
        v0 = (MAIN * NW + wid) * SLAB
        pltpu.async_copy(
            tab_hbm.at[:, pl.ds(v0, SLAB)], buf_v.at[0], rsem[0]
        )
        drain_read(0)
        transpose_slab(0)
        pltpu.async_copy(
            tbuf_v.at[pl.ds(0, SLAB * DIM)],
            out_hbm.at[pl.ds(v0 * DIM, SLAB * DIM)],
            wsem[0],
        )
        drain_write(0)

    @pl.when(wid == 4)
    def _():
        v0 = NSLAB * SLAB
        pltpu.async_copy(tail_hbm, tail_v, rsem[1])
        pltpu.make_async_copy(tail_hbm, tail_v, rsem[1]).wait()
        pltpu.async_copy(
            tail_v, out_hbm.at[pl.ds(v0 * DIM, 64 * DIM)], wsem[1]
        )
        pltpu.make_async_copy(
            tail_v, out_hbm.at[pl.ds(0, 64 * DIM)], wsem[1]
        ).wait()


def kernel(xs, table):
    tail_lin = table[NSLAB * SLAB :].reshape(64 * DIM)
    tab_lin = _format(table.T, tail_lin)
    out_t = _gather(xs.T.astype(jnp.int32), tab_lin.reshape(NVOC, DIM))
    out = jnp.transpose(out_t, (2, 4, 0, 1, 3))
    return out.reshape(BATCH, HIST, DIM)
